# baseline (speedup 1.0000x reference)
import functools

import jax
import jax.numpy as jnp
from jax import lax
from jax.experimental import pallas as pl
from jax.experimental.pallas import tpu as pltpu
from jax.experimental.pallas import tpu_sc as plsc

H = 4096
W = 4096
N = 4194304
M = H * W
PADW = 65536
MP = M + PADW

_info = plsc.get_sparse_core_info()
NC, NS, L = _info.num_cores, _info.num_subcores, _info.num_lanes
NW = NC * NS

CBO = 2048
CBI = CBO // 2
B_PER_TILE = MP // NW
B_CHUNKS = B_PER_TILE // CBO

C2 = 4096
P_PER_TILE = N // NW
P_CHUNKS = P_PER_TILE // C2

MAXX = H - 1
MAXY = W - 1


def _make_build_call():
    mesh = plsc.VectorSubcoreMesh(core_axis_name="c", subcore_axis_name="s")

    @functools.partial(
        pl.kernel,
        mesh=mesh,
        compiler_params=pltpu.CompilerParams(needs_layout_passes=False),
        out_type=jax.ShapeDtypeStruct((MP,), jnp.int32),
        scratch_types=[
            pltpu.VMEM((CBI + 16,), jnp.int32),
            pltpu.VMEM((CBO,), jnp.int32),
        ],
    )
    def build_kernel(a_hbm, p_hbm, av, pv):
        wid = lax.axis_index("s") * NC + lax.axis_index("c")
        obase0 = wid * B_PER_TILE
        ibase0 = wid * (B_PER_TILE // 2)
        lane2 = lax.iota(jnp.int32, L) * 2

        def chunk_body(c, carry):
            obase = obase0 + c * CBO
            ibase = ibase0 + c * CBI
            pltpu.sync_copy(a_hbm.at[pl.ds(ibase, CBI + 16)], av)

            def iv_body(i, carry2):
                o = i * L
                ev = av[pl.ds(o, L)]
                nx = av[pl.ds(o + 1, L)]
                od = lax.shift_right_logical(ev, 16) | (nx << 16)
                pos = lane2 + 2 * o
                plsc.store_scatter(pv, [pos], ev)
                plsc.store_scatter(pv, [pos + 1], od)
                return carry2

            lax.fori_loop(0, CBI // L, iv_body, 0)
            pltpu.sync_copy(pv, p_hbm.at[pl.ds(obase, CBO)])
            return carry

        lax.fori_loop(0, B_CHUNKS, chunk_body, 0)

    return build_kernel


def _f16_decode(h):
    bits = ((h & 0x7FFF) << 13) + (112 << 23)
    bits = bits | ((h & 0x8000) << 16)
    f = plsc.bitcast(bits, jnp.float32)
    return jnp.where((h & 0x7C00) != 0, f, jnp.zeros((L,), jnp.float32))


def _make_lookup_call():
    mesh = plsc.VectorSubcoreMesh(core_axis_name="c", subcore_axis_name="s")

    @functools.partial(
        pl.kernel,
        mesh=mesh,
        compiler_params=pltpu.CompilerParams(needs_layout_passes=False),
        out_type=(
            jax.ShapeDtypeStruct((N,), jnp.float32),
            jax.ShapeDtypeStruct((NW, L), jnp.float32),
        ),
        scratch_types=[
            pltpu.VMEM((C2,), jnp.float32),
            pltpu.VMEM((C2,), jnp.float32),
            pltpu.VMEM((C2,), jnp.int32),
            pltpu.VMEM((C2,), jnp.int32),
            pltpu.VMEM((C2,), jnp.int32),
            pltpu.VMEM((C2,), jnp.int32),
            pltpu.VMEM((C2,), jnp.float32),
            pltpu.VMEM((L,), jnp.float32),
            pltpu.SemaphoreType.DMA,
        ],
    )
    def lookup_kernel(x_hbm, y_hbm, p_hbm, out_hbm, part_hbm,
                      xv, yv, ib0, ib1, g0, g1, ov, accv, gsem):
        wid = lax.axis_index("s") * NC + lax.axis_index("c")
        base0 = wid * P_PER_TILE

        def chunk_body(c, carry):
            base = base0 + c * C2
            pltpu.sync_copy(x_hbm.at[pl.ds(base, C2)], xv)
            pltpu.sync_copy(y_hbm.at[pl.ds(base, C2)], yv)

            def idx_body(i, carry2):
                o = i * L
                xs = xv[pl.ds(o, L)]
                ys = yv[pl.ds(o, L)]
                x1 = jnp.minimum(xs.astype(jnp.int32), MAXX)
                y1 = jnp.minimum(ys.astype(jnp.int32), MAXY)
                b = (x1 << 12) + y1
                ib0[pl.ds(o, L)] = b
                ib1[pl.ds(o, L)] = b + W
                return carry2

            lax.fori_loop(0, C2 // L, idx_body, 0)

            d0 = pltpu.async_copy(p_hbm.at[ib0], g0, gsem)
            d1 = pltpu.async_copy(p_hbm.at[ib1], g1, gsem)
            d0.wait()
            d1.wait()

            def blend_body(i, carry2):
                o = i * L
                xs = xv[pl.ds(o, L)]
                ys = yv[pl.ds(o, L)]
                x1 = jnp.minimum(xs.astype(jnp.int32), MAXX)
                y1 = jnp.minimum(ys.astype(jnp.int32), MAXY)
                t = xs - x1.astype(jnp.float32)
                u = ys - y1.astype(jnp.float32)
                w0 = g0[pl.ds(o, L)]
                w1 = g1[pl.ds(o, L)]
                a = _f16_decode(w0 & 0xFFFF)
                cc = _f16_decode(lax.shift_right_logical(w0, 16))
                b = _f16_decode(w1 & 0xFFFF)
                dd = _f16_decode(lax.shift_right_logical(w1, 16))
                top = a + t * (b - a)
                bot = cc + t * (dd - cc)
                s = top + u * (bot - top)
                s = jnp.minimum(jnp.maximum(s, -1000.0), 0.0)
                ov[pl.ds(o, L)] = s
                return carry2 + s

            acc = lax.fori_loop(0, C2 // L, blend_body, carry)
            pltpu.sync_copy(ov, out_hbm.at[pl.ds(base, C2)])
            return acc

        acc = lax.fori_loop(0, P_CHUNKS, chunk_body, jnp.zeros((L,), jnp.float32))
        accv[...] = acc
        pltpu.sync_copy(accv, part_hbm.at[wid])

    return lookup_kernel


_build_call = _make_build_call()
_lookup_call = _make_lookup_call()


def kernel(x, y, sdf_array):
    sdf16 = sdf_array.astype(jnp.float16).reshape(-1)
    sp = jnp.concatenate([sdf16, jnp.zeros((PADW + 32,), jnp.float16)])
    a32 = jax.lax.bitcast_convert_type(sp.reshape(-1, 2), jnp.int32)
    p = _build_call(a32)
    out_vals, partials = _lookup_call(x, y, p)
    loss = jnp.sum(partials)
    return (loss, out_vals)
